# baseline (speedup 1.0000x reference)
import functools

import jax
import jax.numpy as jnp
from jax import lax
from jax.experimental import pallas as pl
from jax.experimental.pallas import tpu as pltpu

_LANE = 128


def _crf_fwd_kernel(tile_max_ref, lens_ref, logits_ref, exp_trans_ref,
                    trans_max_ref, exp_stop_ref, stop_shift_ref, out_ref,
                    w_ref, c_ref, e_ref, *, start_idx, time_block, unroll):
    bi = pl.program_id(0)
    ti = pl.program_id(1)
    n_t = pl.num_programs(1)
    NL, TB = w_ref.shape

    @pl.when(ti == 0)
    def _():
        row = lax.broadcasted_iota(jnp.int32, (NL, TB), 0)
        w_ref[...] = jnp.where(row == start_idx, jnp.float32(1.0),
                               jnp.float32(0.0))
        c_ref[...] = jnp.zeros((1, TB), jnp.float32)

    base_t = ti * time_block
    tile_max = tile_max_ref[bi]

    @pl.when(base_t < tile_max)
    def _():
        tmax = trans_max_ref[...]
        eb = jnp.transpose(logits_ref[...])
        e_ref[...] = jnp.exp(eb.reshape(time_block, NL, TB) + tmax)
        lens = lens_ref[...]
        expT = exp_trans_ref[...]

        def step(s, carry):
            w, c = carry
            active = lens > (base_t + s)
            z = jnp.sum(w, axis=0, keepdims=True)
            y = jnp.dot(expT, w.astype(jnp.bfloat16),
                        preferred_element_type=jnp.float32)
            scale = e_ref[s] * (1.0 / z)
            w = jnp.where(active, y * scale, w)
            c = c + jnp.where(active, jnp.log(z), jnp.float32(0.0))
            return w, c

        w, c = lax.fori_loop(0, time_block, step, (w_ref[...], c_ref[...]),
                             unroll=unroll)
        w_ref[...] = w
        c_ref[...] = c

    @pl.when(ti == n_t - 1)
    def _():
        s = jnp.sum(w_ref[...] * exp_stop_ref[...], axis=0, keepdims=True)
        out_ref[...] = c_ref[...] + jnp.log(s) + stop_shift_ref[...]


def _crf_forward(logits, lens, transitions, *, time_block=32):
    B, S, NL = logits.shape
    start_idx = NL - 2
    stop_idx = NL - 1

    B_pad = ((B + _LANE - 1) // _LANE) * _LANE
    nb = B_pad // _LANE
    tb_mult = max(d for d in (4, 3, 2, 1) if nb % d == 0)
    TB = _LANE * tb_mult
    n_bt = B_pad // TB

    T = max(1, min(time_block, S))
    S_pad = ((S + T - 1) // T) * T

    logits_p = logits.astype(jnp.float32)
    if B_pad != B or S_pad != S:
        logits_p = jnp.pad(logits_p, ((0, B_pad - B), (0, S_pad - S), (0, 0)))
    logits_p = logits_p.reshape(B_pad, S_pad * NL)

    lens_i32 = lens.astype(jnp.int32)
    if B_pad != B:
        lens_i32 = jnp.pad(lens_i32, (0, B_pad - B))
    lens_p = lens_i32.reshape(1, B_pad)
    tile_max = jnp.max(lens_p.reshape(n_bt, TB), axis=1).astype(jnp.int32)

    trans = transitions.astype(jnp.float32)
    trans_max = jnp.max(trans, axis=1, keepdims=True)
    exp_trans = jnp.exp(trans - trans_max).astype(jnp.bfloat16)
    stop_row = trans[stop_idx, :]
    stop_shift = jnp.max(stop_row).reshape(1, 1)
    exp_stop = jnp.exp(stop_row.reshape(NL, 1) - stop_shift)

    unroll = True if T <= 32 else 8
    kern = functools.partial(_crf_fwd_kernel, start_idx=start_idx,
                             time_block=T, unroll=unroll)

    out = pl.pallas_call(
        kern,
        out_shape=jax.ShapeDtypeStruct((1, B_pad), jnp.float32),
        grid_spec=pltpu.PrefetchScalarGridSpec(
            num_scalar_prefetch=1,
            grid=(n_bt, n_tt := S_pad // T),
            in_specs=[
                pl.BlockSpec((1, TB), lambda bi, ti, tm: (0, bi)),
                pl.BlockSpec((TB, T * NL), lambda bi, ti, tm: (bi, ti)),
                pl.BlockSpec((NL, NL), lambda bi, ti, tm: (0, 0)),
                pl.BlockSpec((NL, 1), lambda bi, ti, tm: (0, 0)),
                pl.BlockSpec((NL, 1), lambda bi, ti, tm: (0, 0)),
                pl.BlockSpec((1, 1), lambda bi, ti, tm: (0, 0)),
            ],
            out_specs=pl.BlockSpec((1, TB), lambda bi, ti, tm: (0, bi)),
            scratch_shapes=[pltpu.VMEM((NL, TB), jnp.float32),
                            pltpu.VMEM((1, TB), jnp.float32),
                            pltpu.VMEM((T, NL, TB), jnp.float32)],
        ),
        compiler_params=pltpu.CompilerParams(
            dimension_semantics=("parallel", "arbitrary")),
    )(tile_max, lens_p, logits_p, exp_trans, trans_max, exp_stop, stop_shift)
    return out[0, :B]


def kernel(logits, lens, transitions):
    return _crf_forward(logits, lens, transitions)


# speedup vs baseline: 1.4580x; 1.4580x over previous
import functools

import jax
import jax.numpy as jnp
from jax import lax
from jax.experimental import pallas as pl
from jax.experimental.pallas import tpu as pltpu

_LANE = 128


def _crf_fwd_kernel(tile_max_ref, lens_ref, logits_ref, exp_trans_ref,
                    trans_max_ref, exp_stop_ref, stop_shift_ref, out_ref,
                    w_ref, c_ref, e_ref, *, start_idx, time_block, unroll):
    bi = pl.program_id(0)
    ti = pl.program_id(1)
    n_t = pl.num_programs(1)
    NL, TB = w_ref.shape

    @pl.when(ti == 0)
    def _():
        row = lax.broadcasted_iota(jnp.int32, (NL, TB), 0)
        w_ref[...] = jnp.where(row == start_idx, jnp.float32(1.0),
                               jnp.float32(0.0))
        c_ref[...] = jnp.zeros((1, TB), jnp.float32)

    base_t = ti * time_block
    tile_max = tile_max_ref[bi]

    @pl.when(base_t < tile_max)
    def _():
        e_ref[...] = jnp.exp(logits_ref[...].astype(jnp.float32)
                             + trans_max_ref[...])
        lens = lens_ref[...]
        expT = exp_trans_ref[...]

        def step(s, carry):
            w, c = carry
            active = lens > (base_t + s)
            z = jnp.sum(w, axis=0, keepdims=True)
            y = jnp.dot(expT, w.astype(jnp.bfloat16),
                        preferred_element_type=jnp.float32)
            scale = e_ref[s] * (1.0 / z)
            w = jnp.where(active, y * scale, w)
            c = c + jnp.where(active, jnp.log(z), jnp.float32(0.0))
            return w, c

        w, c = lax.fori_loop(0, time_block, step, (w_ref[...], c_ref[...]),
                             unroll=unroll)
        w_ref[...] = w
        c_ref[...] = c

    @pl.when(ti == n_t - 1)
    def _():
        s = jnp.sum(w_ref[...] * exp_stop_ref[...], axis=0, keepdims=True)
        out_ref[...] = c_ref[...] + jnp.log(s) + stop_shift_ref[...]


def _crf_forward(logits, lens, transitions, *, time_block=32):
    B, S, NL = logits.shape
    start_idx = NL - 2
    stop_idx = NL - 1

    B_pad = ((B + _LANE - 1) // _LANE) * _LANE
    nb = B_pad // _LANE
    tb_mult = max(d for d in (4, 3, 2, 1) if nb % d == 0)
    TB = _LANE * tb_mult
    n_bt = B_pad // TB

    T = max(1, min(time_block, S))
    S_pad = ((S + T - 1) // T) * T
    n_tt = S_pad // T

    logits_t = jnp.transpose(logits.astype(jnp.bfloat16), (1, 2, 0))
    if B_pad != B or S_pad != S:
        logits_t = jnp.pad(logits_t,
                           ((0, S_pad - S), (0, 0), (0, B_pad - B)))

    lens_i32 = lens.astype(jnp.int32)
    if B_pad != B:
        lens_i32 = jnp.pad(lens_i32, (0, B_pad - B))
    lens_p = lens_i32.reshape(1, B_pad)
    tile_max = jnp.max(lens_p.reshape(n_bt, TB), axis=1).astype(jnp.int32)

    trans = transitions.astype(jnp.float32)
    trans_max = jnp.max(trans, axis=1, keepdims=True)
    exp_trans = jnp.exp(trans - trans_max).astype(jnp.bfloat16)
    stop_row = trans[stop_idx, :]
    stop_shift = jnp.max(stop_row).reshape(1, 1)
    exp_stop = jnp.exp(stop_row.reshape(NL, 1) - stop_shift)

    unroll = True if T <= 32 else 8
    kern = functools.partial(_crf_fwd_kernel, start_idx=start_idx,
                             time_block=T, unroll=unroll)

    out = pl.pallas_call(
        kern,
        out_shape=jax.ShapeDtypeStruct((1, B_pad), jnp.float32),
        grid_spec=pltpu.PrefetchScalarGridSpec(
            num_scalar_prefetch=1,
            grid=(n_bt, n_tt),
            in_specs=[
                pl.BlockSpec((1, TB), lambda bi, ti, tm: (0, bi)),
                pl.BlockSpec((T, NL, TB), lambda bi, ti, tm: (ti, 0, bi)),
                pl.BlockSpec((NL, NL), lambda bi, ti, tm: (0, 0)),
                pl.BlockSpec((NL, 1), lambda bi, ti, tm: (0, 0)),
                pl.BlockSpec((NL, 1), lambda bi, ti, tm: (0, 0)),
                pl.BlockSpec((1, 1), lambda bi, ti, tm: (0, 0)),
            ],
            out_specs=pl.BlockSpec((1, TB), lambda bi, ti, tm: (0, bi)),
            scratch_shapes=[pltpu.VMEM((NL, TB), jnp.float32),
                            pltpu.VMEM((1, TB), jnp.float32),
                            pltpu.VMEM((T, NL, TB), jnp.float32)],
        ),
        compiler_params=pltpu.CompilerParams(
            dimension_semantics=("parallel", "arbitrary")),
    )(tile_max, lens_p, logits_t, exp_trans, trans_max, exp_stop, stop_shift)
    return out[0, :B]


def kernel(logits, lens, transitions):
    return _crf_forward(logits, lens, transitions)
